# baseline (speedup 1.0000x reference)
import functools

import jax
import jax.numpy as jnp
from jax.experimental import pallas as pl
from jax.experimental.pallas import tpu as pltpu

N_BATCH = 16
L = 1024
N_E = 1024
E_DIM = 64
BETA = 0.25
N_ROWS = N_BATCH * L


def _vq_kernel(z_ref, mask_ref, emb_ref, embm2_ref,
               zq_ref, enc_ref, idx_ref, cnt_ref, ssq_ref):
    emb = emb_ref[...]
    zp = jnp.transpose(z_ref[0], (1, 0))
    mask = mask_ref[...]

    zf2 = jnp.sum(zp * zp, axis=1, keepdims=True)
    emb2 = jnp.sum(emb * emb, axis=1)
    mm2 = jax.lax.dot_general(zp, embm2_ref[...], (((1,), (1,)), ((), ())),
                              preferred_element_type=jnp.float32)
    d = (zf2 + emb2) + mm2

    dmin = jnp.min(d, axis=1, keepdims=True)
    ii = jax.lax.broadcasted_iota(jnp.int32, (L, N_E), 1)
    idx = jnp.min(jnp.where(d == dmin, ii, jnp.int32(N_E)), axis=1,
                  keepdims=True)
    idx_ref[...] = idx

    onehot = (ii == idx).astype(jnp.float32)
    enc_ref[...] = onehot

    zq = jax.lax.dot_general(onehot, emb, (((1,), (0,)), ((), ())),
                             preferred_element_type=jnp.float32)
    diff = zq - zp
    zq_ref[0] = jnp.transpose(zp + diff, (1, 0))

    masked = diff * mask
    sq = masked * masked

    ones_row = jnp.ones((1, L), jnp.float32)
    cnt_ref[0] = jax.lax.dot_general(
        ones_row, onehot, (((1,), (0,)), ((), ())),
        preferred_element_type=jnp.float32)
    ssq_ref[0] = jnp.sum(sq, axis=(0, 1), keepdims=True)


@functools.partial(jax.jit, static_argnames=("interpret",))
def kernel(z, mask, emb, interpret=False):
    mask_col = mask.reshape(N_ROWS, 1)
    emb_m2 = emb * jnp.float32(-2.0)

    out_shape = [
        jax.ShapeDtypeStruct((N_BATCH, E_DIM, L), jnp.float32),
        jax.ShapeDtypeStruct((N_ROWS, N_E), jnp.float32),
        jax.ShapeDtypeStruct((N_ROWS, 1), jnp.int32),
        jax.ShapeDtypeStruct((N_BATCH, 1, N_E), jnp.float32),
        jax.ShapeDtypeStruct((N_BATCH, 1, 1), jnp.float32),
    ]
    z_q_out, enc, idx, cnt, ssq = pl.pallas_call(
        _vq_kernel,
        grid=(N_BATCH,),
        in_specs=[
            pl.BlockSpec((1, E_DIM, L), lambda b: (b, 0, 0)),
            pl.BlockSpec((L, 1), lambda b: (b, 0)),
            pl.BlockSpec((N_E, E_DIM), lambda b: (0, 0)),
            pl.BlockSpec((N_E, E_DIM), lambda b: (0, 0)),
        ],
        out_specs=[
            pl.BlockSpec((1, E_DIM, L), lambda b: (b, 0, 0)),
            pl.BlockSpec((L, N_E), lambda b: (b, 0)),
            pl.BlockSpec((L, 1), lambda b: (b, 0)),
            pl.BlockSpec((1, 1, N_E), lambda b: (b, 0, 0)),
            pl.BlockSpec((1, 1, 1), lambda b: (b, 0, 0)),
        ],
        out_shape=out_shape,
        compiler_params=pltpu.CompilerParams(
            dimension_semantics=("parallel",)),
        interpret=interpret,
    )(z, mask_col, emb, emb_m2)

    c = jnp.sum(ssq) / jnp.float32(N_ROWS * E_DIM)
    loss = c + jnp.float32(BETA) * c
    e_mean = jnp.sum(cnt, axis=(0, 1)) / jnp.float32(N_ROWS)
    perplexity = jnp.exp(-jnp.sum(e_mean * jnp.log(e_mean + 1e-10)))
    return (loss, z_q_out, perplexity, enc, idx)


# speedup vs baseline: 1.0347x; 1.0347x over previous
import functools

import jax
import jax.numpy as jnp
from jax.experimental import pallas as pl
from jax.experimental.pallas import tpu as pltpu

N_BATCH = 16
L = 1024
N_E = 1024
E_DIM = 64
BETA = 0.25
N_ROWS = N_BATCH * L


def _vq_kernel(z_ref, mask_ref, emb_ref, embm2_ref,
               zq_ref, enc_ref, idx_ref, loss_ref, perp_ref,
               cnt_ref, ssq_ref, emb2_ref):
    b = pl.program_id(0)

    emb = emb_ref[...]

    @pl.when(b == 0)
    def _init():
        cnt_ref[...] = jnp.zeros_like(cnt_ref)
        ssq_ref[...] = jnp.zeros_like(ssq_ref)
        emb2_ref[...] = jnp.sum(emb * emb, axis=1, keepdims=True).T

    zp = jnp.transpose(z_ref[0], (1, 0))
    mask = mask_ref[...]

    zf2 = jnp.sum(zp * zp, axis=1, keepdims=True)
    emb2 = emb2_ref[...]
    mm2 = jax.lax.dot_general(zp, embm2_ref[...], (((1,), (1,)), ((), ())),
                              preferred_element_type=jnp.float32)
    d = (zf2 + emb2) + mm2

    dmin = jnp.min(d, axis=1, keepdims=True)
    ii = jax.lax.broadcasted_iota(jnp.int32, (L, N_E), 1)
    idx = jnp.min(jnp.where(d == dmin, ii, jnp.int32(N_E)), axis=1,
                  keepdims=True)
    idx_ref[...] = idx

    onehot = (ii == idx).astype(jnp.float32)
    enc_ref[...] = onehot

    zq = jax.lax.dot_general(onehot, emb, (((1,), (0,)), ((), ())),
                             preferred_element_type=jnp.float32)
    diff = zq - zp
    zq_ref[0] = jnp.transpose(zp + diff, (1, 0))

    masked = diff * mask
    sq = masked * masked

    ones_row = jnp.ones((1, L), jnp.float32)
    cnt_ref[...] += jax.lax.dot_general(
        ones_row, onehot, (((1,), (0,)), ((), ())),
        preferred_element_type=jnp.float32)
    ssq_ref[...] += jnp.sum(sq, axis=(0, 1), keepdims=True)

    @pl.when(b == N_BATCH - 1)
    def _finish():
        c = ssq_ref[...] / jnp.float32(N_ROWS * E_DIM)
        loss_ref[...] = c + jnp.float32(BETA) * c
        e_mean = cnt_ref[...] / jnp.float32(N_ROWS)
        ent = jnp.sum(e_mean * jnp.log(e_mean + 1e-10), axis=(0, 1),
                      keepdims=True)
        perp_ref[...] = jnp.exp(-ent)


@functools.partial(jax.jit, static_argnames=("interpret",))
def kernel(z, mask, emb, interpret=False):
    mask_col = mask.reshape(N_ROWS, 1)
    emb_m2 = emb * jnp.float32(-2.0)

    out_shape = [
        jax.ShapeDtypeStruct((N_BATCH, E_DIM, L), jnp.float32),
        jax.ShapeDtypeStruct((N_ROWS, N_E), jnp.float32),
        jax.ShapeDtypeStruct((N_ROWS, 1), jnp.int32),
        jax.ShapeDtypeStruct((1, 1), jnp.float32),
        jax.ShapeDtypeStruct((1, 1), jnp.float32),
    ]
    z_q_out, enc, idx, loss2, perp2 = pl.pallas_call(
        _vq_kernel,
        grid=(N_BATCH,),
        in_specs=[
            pl.BlockSpec((1, E_DIM, L), lambda b: (b, 0, 0)),
            pl.BlockSpec((L, 1), lambda b: (b, 0)),
            pl.BlockSpec((N_E, E_DIM), lambda b: (0, 0)),
            pl.BlockSpec((N_E, E_DIM), lambda b: (0, 0)),
        ],
        out_specs=[
            pl.BlockSpec((1, E_DIM, L), lambda b: (b, 0, 0)),
            pl.BlockSpec((L, N_E), lambda b: (b, 0)),
            pl.BlockSpec((L, 1), lambda b: (b, 0)),
            pl.BlockSpec((1, 1), lambda b: (0, 0)),
            pl.BlockSpec((1, 1), lambda b: (0, 0)),
        ],
        out_shape=out_shape,
        scratch_shapes=[
            pltpu.VMEM((1, N_E), jnp.float32),
            pltpu.VMEM((1, 1), jnp.float32),
            pltpu.VMEM((1, N_E), jnp.float32),
        ],
        compiler_params=pltpu.CompilerParams(
            dimension_semantics=("arbitrary",)),
        interpret=interpret,
    )(z, mask_col, emb, emb_m2)

    loss = loss2[0, 0]
    perplexity = perp2[0, 0]
    return (loss, z_q_out, perplexity, enc, idx)


# speedup vs baseline: 1.0409x; 1.0060x over previous
import functools

import jax
import jax.numpy as jnp
from jax.experimental import pallas as pl
from jax.experimental.pallas import tpu as pltpu

N_BATCH = 16
L = 1024
N_E = 1024
E_DIM = 64
BETA = 0.25
N_ROWS = N_BATCH * L


def _vq_kernel(z_ref, mask_ref, emb_ref,
               zq_ref, enc_ref, idx_ref, loss_ref, perp_ref,
               cnt_ref, ssq_ref, emb2_ref, embm2_ref):
    b = pl.program_id(0)

    emb = emb_ref[...]

    @pl.when(b == 0)
    def _init():
        cnt_ref[...] = jnp.zeros_like(cnt_ref)
        ssq_ref[...] = jnp.zeros_like(ssq_ref)
        emb2_ref[...] = jnp.sum(emb * emb, axis=1, keepdims=True).T
        embm2_ref[...] = emb * jnp.float32(-2.0)

    zp = jnp.transpose(z_ref[0], (1, 0))
    mask = mask_ref[...]

    zf2 = jnp.sum(zp * zp, axis=1, keepdims=True)
    emb2 = emb2_ref[...]
    mm2 = jax.lax.dot_general(zp, embm2_ref[...], (((1,), (1,)), ((), ())),
                              preferred_element_type=jnp.float32)
    d = (zf2 + emb2) + mm2

    dmin = jnp.min(d, axis=1, keepdims=True)
    ii = jax.lax.broadcasted_iota(jnp.int32, (L, N_E), 1)
    idx = jnp.min(jnp.where(d == dmin, ii, jnp.int32(N_E)), axis=1,
                  keepdims=True)
    idx_ref[...] = idx

    onehot = (ii == idx).astype(jnp.float32)
    enc_ref[...] = onehot

    zq = jax.lax.dot_general(onehot, emb, (((1,), (0,)), ((), ())),
                             preferred_element_type=jnp.float32)
    diff = zq - zp
    zq_ref[0] = jnp.transpose(zp + diff, (1, 0))

    masked = diff * mask
    sq = masked * masked

    ones_row = jnp.ones((1, L), jnp.float32)
    cnt_ref[...] += jax.lax.dot_general(
        ones_row, onehot, (((1,), (0,)), ((), ())),
        preferred_element_type=jnp.float32)
    rowsq = jax.lax.dot_general(
        sq, jnp.ones((E_DIM, 1), jnp.float32), (((1,), (0,)), ((), ())),
        preferred_element_type=jnp.float32)
    ssq_ref[...] += jax.lax.dot_general(
        ones_row, rowsq, (((1,), (0,)), ((), ())),
        preferred_element_type=jnp.float32)

    @pl.when(b == N_BATCH - 1)
    def _finish():
        c = ssq_ref[...] / jnp.float32(N_ROWS * E_DIM)
        loss_ref[...] = c + jnp.float32(BETA) * c
        e_mean = cnt_ref[...] / jnp.float32(N_ROWS)
        ent = jnp.sum(e_mean * jnp.log(e_mean + 1e-10), axis=(0, 1),
                      keepdims=True)
        perp_ref[...] = jnp.exp(-ent)


@functools.partial(jax.jit, static_argnames=("interpret",))
def kernel(z, mask, emb, interpret=False):
    mask_col = mask.reshape(N_ROWS, 1)

    out_shape = [
        jax.ShapeDtypeStruct((N_BATCH, E_DIM, L), jnp.float32),
        jax.ShapeDtypeStruct((N_ROWS, N_E), jnp.float32),
        jax.ShapeDtypeStruct((N_ROWS, 1), jnp.int32),
        jax.ShapeDtypeStruct((1, 1), jnp.float32),
        jax.ShapeDtypeStruct((1, 1), jnp.float32),
    ]
    z_q_out, enc, idx, loss2, perp2 = pl.pallas_call(
        _vq_kernel,
        grid=(N_BATCH,),
        in_specs=[
            pl.BlockSpec((1, E_DIM, L), lambda b: (b, 0, 0)),
            pl.BlockSpec((L, 1), lambda b: (b, 0)),
            pl.BlockSpec((N_E, E_DIM), lambda b: (0, 0)),
        ],
        out_specs=[
            pl.BlockSpec((1, E_DIM, L), lambda b: (b, 0, 0)),
            pl.BlockSpec((L, N_E), lambda b: (b, 0)),
            pl.BlockSpec((L, 1), lambda b: (b, 0)),
            pl.BlockSpec((1, 1), lambda b: (0, 0)),
            pl.BlockSpec((1, 1), lambda b: (0, 0)),
        ],
        out_shape=out_shape,
        scratch_shapes=[
            pltpu.VMEM((1, N_E), jnp.float32),
            pltpu.VMEM((1, 1), jnp.float32),
            pltpu.VMEM((1, N_E), jnp.float32),
            pltpu.VMEM((N_E, E_DIM), jnp.float32),
        ],
        compiler_params=pltpu.CompilerParams(
            dimension_semantics=("arbitrary",)),
        interpret=interpret,
    )(z, mask_col, emb)

    loss = loss2[0, 0]
    perplexity = perp2[0, 0]
    return (loss, z_q_out, perplexity, enc, idx)


# speedup vs baseline: 1.0424x; 1.0014x over previous
import functools

import jax
import jax.numpy as jnp
from jax.experimental import pallas as pl
from jax.experimental.pallas import tpu as pltpu

N_BATCH = 16
L = 1024
N_E = 1024
E_DIM = 64
BETA = 0.25
N_ROWS = N_BATCH * L


def _vq_kernel(z_ref, mask_ref, emb_ref,
               zq_ref, enc_ref, idx_ref, loss_ref, perp_ref,
               cnt_ref, ssq_ref, emb2_ref, embm2_ref):
    b = pl.program_id(0)

    emb = emb_ref[...]

    @pl.when(b == 0)
    def _init():
        cnt_ref[...] = jnp.zeros_like(cnt_ref)
        ssq_ref[...] = jnp.zeros_like(ssq_ref)
        emb2_ref[...] = jnp.sum(emb * emb, axis=1, keepdims=True).T
        embm2_ref[...] = emb * jnp.float32(-2.0)

    zp = jnp.transpose(z_ref[0], (1, 0))
    mask = mask_ref[...]

    zf2 = jnp.sum(zp * zp, axis=1, keepdims=True)
    emb2 = emb2_ref[...]
    mm2 = jax.lax.dot_general(zp, embm2_ref[...], (((1,), (1,)), ((), ())),
                              preferred_element_type=jnp.float32)
    d = (zf2 + emb2) + mm2

    dmin = jnp.min(d, axis=1, keepdims=True)
    ii = jax.lax.broadcasted_iota(jnp.int32, (L, N_E), 1)
    idx = jnp.min(jnp.where(d == dmin, ii, jnp.int32(N_E)), axis=1,
                  keepdims=True)
    idx_ref[...] = idx

    onehot = (ii == idx).astype(jnp.float32)
    enc_ref[...] = onehot

    zq = jax.lax.dot_general(onehot, emb, (((1,), (0,)), ((), ())),
                             preferred_element_type=jnp.float32)
    diff = zq - zp
    zq_ref[0] = jnp.transpose(zp + diff, (1, 0))

    masked = diff * mask
    sq = masked * masked

    ones_row = jnp.ones((1, L), jnp.float32)
    cnt_ref[...] += jax.lax.dot_general(
        ones_row, onehot, (((1,), (0,)), ((), ())),
        preferred_element_type=jnp.float32)
    rowsq = jax.lax.dot_general(
        sq, jnp.ones((E_DIM, 1), jnp.float32), (((1,), (0,)), ((), ())),
        preferred_element_type=jnp.float32)
    ssq_ref[...] += jax.lax.dot_general(
        ones_row, rowsq, (((1,), (0,)), ((), ())),
        preferred_element_type=jnp.float32)

    @pl.when(b == N_BATCH - 1)
    def _finish():
        c = ssq_ref[...] / jnp.float32(N_ROWS * E_DIM)
        loss_ref[...] = c + jnp.float32(BETA) * c
        e_mean = cnt_ref[...] / jnp.float32(N_ROWS)
        ent = jnp.sum(e_mean * jnp.log(e_mean + 1e-10), axis=(0, 1),
                      keepdims=True)
        perp_ref[...] = jnp.exp(-ent)


@functools.partial(jax.jit, static_argnames=("interpret",))
def kernel(z, mask, emb, interpret=False):
    mask_col = mask.reshape(N_ROWS, 1)

    out_shape = [
        jax.ShapeDtypeStruct((N_BATCH, E_DIM, L), jnp.float32),
        jax.ShapeDtypeStruct((N_ROWS, N_E), jnp.float32),
        jax.ShapeDtypeStruct((N_ROWS, 1), jnp.int32),
        jax.ShapeDtypeStruct((1, 1), jnp.float32),
        jax.ShapeDtypeStruct((1, 1), jnp.float32),
    ]
    z_q_out, enc, idx, loss2, perp2 = pl.pallas_call(
        _vq_kernel,
        grid=(N_BATCH,),
        in_specs=[
            pl.BlockSpec((1, E_DIM, L), lambda b: (b, 0, 0)),
            pl.BlockSpec((L, 1), lambda b: (b, 0)),
            pl.BlockSpec((N_E, E_DIM), lambda b: (0, 0)),
        ],
        out_specs=[
            pl.BlockSpec((1, E_DIM, L), lambda b: (b, 0, 0)),
            pl.BlockSpec((L, N_E), lambda b: (b, 0)),
            pl.BlockSpec((L, 1), lambda b: (b, 0)),
            pl.BlockSpec((1, 1), lambda b: (0, 0)),
            pl.BlockSpec((1, 1), lambda b: (0, 0)),
        ],
        out_shape=out_shape,
        scratch_shapes=[
            pltpu.VMEM((1, N_E), jnp.float32),
            pltpu.VMEM((1, 1), jnp.float32),
            pltpu.VMEM((1, N_E), jnp.float32),
            pltpu.VMEM((N_E, E_DIM), jnp.float32),
        ],
        compiler_params=pltpu.CompilerParams(
            dimension_semantics=("arbitrary",)),
        interpret=interpret,
    )(z, mask_col, emb)

    loss = loss2.reshape(())
    perplexity = perp2.reshape(())
    return (loss, z_q_out, perplexity, enc, idx)


# speedup vs baseline: 1.1822x; 1.1342x over previous
import functools

import jax
import jax.numpy as jnp
from jax.experimental import pallas as pl
from jax.experimental.pallas import tpu as pltpu

N_BATCH = 16
L = 1024
N_E = 1024
E_DIM = 64
BETA = 0.25
N_ROWS = N_BATCH * L


def _vq_kernel(z_ref, mask_ref, emb_ref,
               zq_ref, enc_ref, idx_ref, loss_ref, perp_ref,
               cnt_ref, ssq_ref, emb2_ref, embm2_ref):
    b = pl.program_id(0)

    emb = emb_ref[...]

    @pl.when(b == 0)
    def _init():
        cnt_ref[...] = jnp.zeros_like(cnt_ref)
        ssq_ref[...] = jnp.zeros_like(ssq_ref)
        emb2_ref[...] = jnp.sum(emb * emb, axis=1, keepdims=True).T
        embm2_ref[...] = emb * jnp.float32(-2.0)

    z_cl = z_ref[0]
    mask = mask_ref[0]

    zf2 = jnp.sum(z_cl * z_cl, axis=0, keepdims=True)
    zf2_col = zf2.reshape(L, 1)
    emb2 = emb2_ref[...]
    mm2 = jax.lax.dot_general(z_cl, embm2_ref[...], (((0,), (1,)), ((), ())),
                              preferred_element_type=jnp.float32)
    d = (zf2_col + emb2) + mm2

    dmin = jnp.min(d, axis=1, keepdims=True)
    ii = jax.lax.broadcasted_iota(jnp.int32, (L, N_E), 1)
    idx = jnp.min(jnp.where(d == dmin, ii, jnp.int32(N_E)), axis=1,
                  keepdims=True)
    idx_ref[...] = idx

    onehot = (ii == idx).astype(jnp.float32)
    enc_ref[...] = onehot

    zq_cl = jax.lax.dot_general(emb, onehot, (((0,), (1,)), ((), ())),
                                preferred_element_type=jnp.float32)
    diff = zq_cl - z_cl
    zq_ref[0] = z_cl + diff

    masked = diff * mask
    sq = masked * masked

    ones_row = jnp.ones((1, L), jnp.float32)
    cnt_ref[...] += jax.lax.dot_general(
        ones_row, onehot, (((1,), (0,)), ((), ())),
        preferred_element_type=jnp.float32)
    colsq = jax.lax.dot_general(
        jnp.ones((1, E_DIM), jnp.float32), sq, (((1,), (0,)), ((), ())),
        preferred_element_type=jnp.float32)
    ssq_ref[...] += jax.lax.dot_general(
        colsq, jnp.ones((L, 1), jnp.float32), (((1,), (0,)), ((), ())),
        preferred_element_type=jnp.float32)

    @pl.when(b == N_BATCH - 1)
    def _finish():
        c = ssq_ref[...] / jnp.float32(N_ROWS * E_DIM)
        loss_ref[...] = c + jnp.float32(BETA) * c
        e_mean = cnt_ref[...] / jnp.float32(N_ROWS)
        ent = jnp.sum(e_mean * jnp.log(e_mean + 1e-10), axis=(0, 1),
                      keepdims=True)
        perp_ref[...] = jnp.exp(-ent)


@functools.partial(jax.jit, static_argnames=("interpret",))
def kernel(z, mask, emb, interpret=False):
    mask_rows = mask.reshape(N_BATCH, 1, L)

    out_shape = [
        jax.ShapeDtypeStruct((N_BATCH, E_DIM, L), jnp.float32),
        jax.ShapeDtypeStruct((N_ROWS, N_E), jnp.float32),
        jax.ShapeDtypeStruct((N_ROWS, 1), jnp.int32),
        jax.ShapeDtypeStruct((1, 1), jnp.float32),
        jax.ShapeDtypeStruct((1, 1), jnp.float32),
    ]
    z_q_out, enc, idx, loss2, perp2 = pl.pallas_call(
        _vq_kernel,
        grid=(N_BATCH,),
        in_specs=[
            pl.BlockSpec((1, E_DIM, L), lambda b: (b, 0, 0)),
            pl.BlockSpec((1, 1, L), lambda b: (b, 0, 0)),
            pl.BlockSpec((N_E, E_DIM), lambda b: (0, 0)),
        ],
        out_specs=[
            pl.BlockSpec((1, E_DIM, L), lambda b: (b, 0, 0)),
            pl.BlockSpec((L, N_E), lambda b: (b, 0)),
            pl.BlockSpec((L, 1), lambda b: (b, 0)),
            pl.BlockSpec((1, 1), lambda b: (0, 0)),
            pl.BlockSpec((1, 1), lambda b: (0, 0)),
        ],
        out_shape=out_shape,
        scratch_shapes=[
            pltpu.VMEM((1, N_E), jnp.float32),
            pltpu.VMEM((1, 1), jnp.float32),
            pltpu.VMEM((1, N_E), jnp.float32),
            pltpu.VMEM((N_E, E_DIM), jnp.float32),
        ],
        compiler_params=pltpu.CompilerParams(
            dimension_semantics=("arbitrary",)),
        interpret=interpret,
    )(z, mask_rows, emb)

    loss = loss2.reshape(())
    perplexity = perp2.reshape(())
    return (loss, z_q_out, perplexity, enc, idx)
